# baseline (speedup 1.0000x reference)
import functools

import jax
import jax.numpy as jnp
from jax import lax
from jax.experimental import pallas as pl
from jax.experimental.pallas import tpu as pltpu
from jax.experimental.pallas import tpu_sc as plsc

N_USERS = 100000
EMBED_DIM = 64
B = 4096

NUM_CORES = 2
NUM_SUBCORES = 16
LANES = 16
NW = NUM_CORES * NUM_SUBCORES
PER_W = B // NW
ROWS_PER_W = 2 * PER_W
CHUNKS = ROWS_PER_W // LANES


def _sc_gather_body(users_hbm, items_hbm, table_hbm, u_out, it_out,
                    idx_v, rows_v, sem):
    wid = lax.axis_index("s") * NUM_CORES + lax.axis_index("c")
    base = wid * PER_W
    pltpu.sync_copy(users_hbm.at[pl.ds(base, PER_W)], idx_v.at[pl.ds(0, PER_W)])
    pltpu.sync_copy(items_hbm.at[pl.ds(base, PER_W)],
                    idx_v.at[pl.ds(PER_W, PER_W)])
    lane = lax.iota(jnp.int32, LANES)

    def chunk_body(c, _):
        vec = idx_v[pl.ds(c * LANES, LANES)]
        for j in range(LANES):
            r = jnp.sum(jnp.where(lane == j, vec, 0))
            pltpu.async_copy(table_hbm.at[pl.ds(r, 1)],
                             rows_v.at[pl.ds(c * LANES + j, 1)], sem)
        return 0

    lax.fori_loop(0, CHUNKS, chunk_body, 0)
    pltpu.make_async_copy(table_hbm.at[pl.ds(0, ROWS_PER_W)], rows_v, sem).wait()
    pltpu.sync_copy(rows_v.at[pl.ds(0, PER_W)], u_out.at[pl.ds(base, PER_W)])
    pltpu.sync_copy(rows_v.at[pl.ds(PER_W, PER_W)],
                    it_out.at[pl.ds(base, PER_W)])


def _make_sc_gather():
    return functools.partial(
        pl.kernel,
        mesh=plsc.VectorSubcoreMesh(core_axis_name="c", subcore_axis_name="s"),
        out_type=(
            jax.ShapeDtypeStruct((B, EMBED_DIM), jnp.float32),
            jax.ShapeDtypeStruct((B, EMBED_DIM), jnp.float32),
        ),
        scratch_types=[
            pltpu.VMEM((ROWS_PER_W,), jnp.int32),
            pltpu.VMEM((ROWS_PER_W, EMBED_DIM), jnp.float32),
            pltpu.SemaphoreType.DMA,
        ],
        compiler_params=pltpu.CompilerParams(needs_layout_passes=False),
    )(_sc_gather_body)


def _mlp_body(u_ref, i_ref, w1_ref, b1_ref, w2_ref, b2_ref,
              w3_ref, b3_ref, w4_ref, out_ref):
    u = u_ref[...].astype(jnp.float32)
    it = i_ref[...].astype(jnp.float32)
    x = (u @ w1_ref[:EMBED_DIM, :] + it @ w1_ref[EMBED_DIM:, :]
         + b1_ref[...].reshape(1, -1))
    x = jnp.maximum(x, 0.0)
    x = jnp.maximum(x @ w2_ref[...] + b2_ref[...].reshape(1, -1), 0.0)
    x = jnp.maximum(x @ w3_ref[...] + b3_ref[...].reshape(1, -1), 0.0)
    out_ref[...] = x @ w4_ref[...]


TCOL = 512
TGRID = -(-N_USERS // TCOL)


def _transpose_body(tt_ref, out_ref):
    a = tt_ref[...]
    row = lax.broadcasted_iota(jnp.int32, (EMBED_DIM, EMBED_DIM), 0)
    col = lax.broadcasted_iota(jnp.int32, (EMBED_DIM, EMBED_DIM), 1)
    eye = (row == col).astype(jnp.float32)
    out_ref[...] = lax.dot_general(a, eye, (((0,), (0,)), ((), ())),
                                   preferred_element_type=jnp.float32)


def _relayout_table(table_user):
    tt = table_user.T
    return pl.pallas_call(
        _transpose_body,
        grid=(TGRID,),
        in_specs=[pl.BlockSpec((EMBED_DIM, TCOL), lambda i: (0, i))],
        out_specs=pl.BlockSpec((TCOL, EMBED_DIM), lambda i: (i, 0)),
        out_shape=jax.ShapeDtypeStruct((N_USERS, EMBED_DIM), jnp.float32),
        compiler_params=pltpu.CompilerParams(fuse_transposed_lhs_in_matmul=True),
    )(tt)


def kernel(users, items, table_user, W1, b1, W2, b2, W3, b3, W4):
    tbl = _relayout_table(table_user)
    u, it = _make_sc_gather()(users.astype(jnp.int32), items.astype(jnp.int32),
                              tbl)
    score = pl.pallas_call(
        _mlp_body,
        out_shape=jax.ShapeDtypeStruct((B, 1), jnp.float32),
    )(u, it, W1, b1, W2, b2, W3, b3, W4)
    return score


# speedup vs baseline: 1.9847x; 1.9847x over previous
import functools

import jax
import jax.numpy as jnp
from jax import lax
from jax.experimental import pallas as pl
from jax.experimental.pallas import tpu as pltpu
from jax.experimental.pallas import tpu_sc as plsc

N_USERS = 100000
EMBED_DIM = 64
B = 4096

NUM_CORES = 2
NUM_SUBCORES = 16
LANES = 16
NW = NUM_CORES * NUM_SUBCORES
PER_W = B // NW
ROWS_PER_W = 2 * PER_W
CHUNKS = ROWS_PER_W // LANES


def _sc_gather_body(users_hbm, items_hbm, table_hbm, u_out, it_out,
                    idx_v, rows_v, sem):
    wid = lax.axis_index("s") * NUM_CORES + lax.axis_index("c")
    base = wid * PER_W
    pltpu.sync_copy(users_hbm.at[pl.ds(base, PER_W)], idx_v.at[pl.ds(0, PER_W)])
    pltpu.sync_copy(items_hbm.at[pl.ds(base, PER_W)],
                    idx_v.at[pl.ds(PER_W, PER_W)])
    lane = lax.iota(jnp.int32, LANES)

    def chunk_body(c, _):
        vec = idx_v[pl.ds(c * LANES, LANES)]
        for j in range(LANES):
            r = jnp.sum(jnp.where(lane == j, vec, 0))
            pltpu.async_copy(table_hbm.at[pl.ds(r, 1)],
                             rows_v.at[pl.ds(c * LANES + j, 1)], sem)
        return 0

    lax.fori_loop(0, CHUNKS, chunk_body, 0)
    pltpu.make_async_copy(table_hbm.at[pl.ds(0, ROWS_PER_W)], rows_v, sem).wait()
    pltpu.sync_copy(rows_v.at[pl.ds(0, PER_W)], u_out.at[pl.ds(base, PER_W)])
    pltpu.sync_copy(rows_v.at[pl.ds(PER_W, PER_W)],
                    it_out.at[pl.ds(base, PER_W)])


def _make_sc_gather():
    return functools.partial(
        pl.kernel,
        mesh=plsc.VectorSubcoreMesh(core_axis_name="c", subcore_axis_name="s"),
        out_type=(
            jax.ShapeDtypeStruct((B, EMBED_DIM), jnp.float32),
            jax.ShapeDtypeStruct((B, EMBED_DIM), jnp.float32),
        ),
        scratch_types=[
            pltpu.VMEM((ROWS_PER_W,), jnp.int32),
            pltpu.VMEM((ROWS_PER_W, EMBED_DIM), jnp.float32),
            pltpu.SemaphoreType.DMA,
        ],
        compiler_params=pltpu.CompilerParams(needs_layout_passes=False),
    )(_sc_gather_body)


def _mlp_body(u_ref, i_ref, w1_ref, b1_ref, w2_ref, b2_ref,
              w3_ref, b3_ref, w4_ref, out_ref):
    u = u_ref[...].astype(jnp.float32)
    it = i_ref[...].astype(jnp.float32)
    x = (u @ w1_ref[:EMBED_DIM, :] + it @ w1_ref[EMBED_DIM:, :]
         + b1_ref[...].reshape(1, -1))
    x = jnp.maximum(x, 0.0)
    x = jnp.maximum(x @ w2_ref[...] + b2_ref[...].reshape(1, -1), 0.0)
    x = jnp.maximum(x @ w3_ref[...] + b3_ref[...].reshape(1, -1), 0.0)
    out_ref[...] = x @ w4_ref[...]


TCOL = 2048
TGRID = -(-N_USERS // TCOL)


def _transpose_body(tt_ref, out_ref):
    a = tt_ref[...]
    row = lax.broadcasted_iota(jnp.int32, (EMBED_DIM, EMBED_DIM), 0)
    col = lax.broadcasted_iota(jnp.int32, (EMBED_DIM, EMBED_DIM), 1)
    eye = (row == col).astype(jnp.float32)
    out_ref[...] = lax.dot_general(a, eye, (((0,), (0,)), ((), ())),
                                   preferred_element_type=jnp.float32)


def _relayout_table(table_user):
    tt = table_user.T
    return pl.pallas_call(
        _transpose_body,
        grid=(TGRID,),
        in_specs=[pl.BlockSpec((EMBED_DIM, TCOL), lambda i: (0, i))],
        out_specs=pl.BlockSpec((TCOL, EMBED_DIM), lambda i: (i, 0)),
        out_shape=jax.ShapeDtypeStruct((N_USERS, EMBED_DIM), jnp.float32),
        compiler_params=pltpu.CompilerParams(fuse_transposed_lhs_in_matmul=True),
    )(tt)


def kernel(users, items, table_user, W1, b1, W2, b2, W3, b3, W4):
    tbl = _relayout_table(table_user)
    u, it = _make_sc_gather()(users.astype(jnp.int32), items.astype(jnp.int32),
                              tbl)
    score = pl.pallas_call(
        _mlp_body,
        out_shape=jax.ShapeDtypeStruct((B, 1), jnp.float32),
    )(u, it, W1, b1, W2, b2, W3, b3, W4)
    return score


# speedup vs baseline: 2.6352x; 1.3278x over previous
import functools

import jax
import jax.numpy as jnp
from jax import lax
from jax.experimental import pallas as pl
from jax.experimental.pallas import tpu as pltpu
from jax.experimental.pallas import tpu_sc as plsc

N_USERS = 100000
EMBED_DIM = 64
B = 4096

NUM_CORES = 2
NUM_SUBCORES = 16
LANES = 16
NW = NUM_CORES * NUM_SUBCORES
PER_W = B // NW
ROWS_PER_W = 2 * PER_W
CHUNKS = ROWS_PER_W // LANES


def _sc_gather_body(users_hbm, items_hbm, table_hbm, u_out, it_out,
                    idx_v, rows_v, sem):
    wid = lax.axis_index("s") * NUM_CORES + lax.axis_index("c")
    base = wid * PER_W
    pltpu.sync_copy(users_hbm.at[pl.ds(base, PER_W)], idx_v.at[pl.ds(0, PER_W)])
    pltpu.sync_copy(items_hbm.at[pl.ds(base, PER_W)],
                    idx_v.at[pl.ds(PER_W, PER_W)])
    lane = lax.iota(jnp.int32, LANES)

    def chunk_body(c, _):
        vec = idx_v[pl.ds(c * LANES, LANES)]
        for j in range(LANES):
            r = jnp.sum(jnp.where(lane == j, vec, 0))
            pltpu.async_copy(table_hbm.at[pl.ds(r, 1)],
                             rows_v.at[pl.ds(c * LANES + j, 1)], sem)
        return 0

    lax.fori_loop(0, CHUNKS, chunk_body, 0)
    pltpu.make_async_copy(table_hbm.at[pl.ds(0, ROWS_PER_W)], rows_v, sem).wait()
    pltpu.sync_copy(rows_v.at[pl.ds(0, PER_W)], u_out.at[pl.ds(base, PER_W)])
    pltpu.sync_copy(rows_v.at[pl.ds(PER_W, PER_W)],
                    it_out.at[pl.ds(base, PER_W)])


def _make_sc_gather():
    return functools.partial(
        pl.kernel,
        mesh=plsc.VectorSubcoreMesh(core_axis_name="c", subcore_axis_name="s"),
        out_type=(
            jax.ShapeDtypeStruct((B, EMBED_DIM), jnp.float32),
            jax.ShapeDtypeStruct((B, EMBED_DIM), jnp.float32),
        ),
        scratch_types=[
            pltpu.VMEM((ROWS_PER_W,), jnp.int32),
            pltpu.VMEM((ROWS_PER_W, EMBED_DIM), jnp.float32),
            pltpu.SemaphoreType.DMA,
        ],
        compiler_params=pltpu.CompilerParams(needs_layout_passes=False),
    )(_sc_gather_body)


def _mlp_body(u_ref, i_ref, w1_ref, b1_ref, w2_ref, b2_ref,
              w3_ref, b3_ref, w4_ref, out_ref):
    u = u_ref[...].astype(jnp.float32)
    it = i_ref[...].astype(jnp.float32)
    x = (u @ w1_ref[:EMBED_DIM, :] + it @ w1_ref[EMBED_DIM:, :]
         + b1_ref[...].reshape(1, -1))
    x = jnp.maximum(x, 0.0)
    x = jnp.maximum(x @ w2_ref[...] + b2_ref[...].reshape(1, -1), 0.0)
    x = jnp.maximum(x @ w3_ref[...] + b3_ref[...].reshape(1, -1), 0.0)
    out_ref[...] = x @ w4_ref[...]


TCOL = 8192
TGRID = -(-N_USERS // TCOL)


def _transpose_body(tt_ref, out_ref):
    a = tt_ref[...]
    row = lax.broadcasted_iota(jnp.int32, (EMBED_DIM, EMBED_DIM), 0)
    col = lax.broadcasted_iota(jnp.int32, (EMBED_DIM, EMBED_DIM), 1)
    eye = (row == col).astype(jnp.float32)
    out_ref[...] = lax.dot_general(a, eye, (((0,), (0,)), ((), ())),
                                   preferred_element_type=jnp.float32)


def _relayout_table(table_user):
    tt = table_user.T
    return pl.pallas_call(
        _transpose_body,
        grid=(TGRID,),
        in_specs=[pl.BlockSpec((EMBED_DIM, TCOL), lambda i: (0, i))],
        out_specs=pl.BlockSpec((TCOL, EMBED_DIM), lambda i: (i, 0)),
        out_shape=jax.ShapeDtypeStruct((N_USERS, EMBED_DIM), jnp.float32),
        compiler_params=pltpu.CompilerParams(fuse_transposed_lhs_in_matmul=True),
    )(tt)


def kernel(users, items, table_user, W1, b1, W2, b2, W3, b3, W4):
    tbl = _relayout_table(table_user)
    u, it = _make_sc_gather()(users.astype(jnp.int32), items.astype(jnp.int32),
                              tbl)
    score = pl.pallas_call(
        _mlp_body,
        out_shape=jax.ShapeDtypeStruct((B, 1), jnp.float32),
    )(u, it, W1, b1, W2, b2, W3, b3, W4)
    return score


# speedup vs baseline: 2.7493x; 1.0433x over previous
import functools

import jax
import jax.numpy as jnp
from jax import lax
from jax.experimental import pallas as pl
from jax.experimental.pallas import tpu as pltpu
from jax.experimental.pallas import tpu_sc as plsc

N_USERS = 100000
EMBED_DIM = 64
B = 4096

NUM_CORES = 2
NUM_SUBCORES = 16
LANES = 16
NW = NUM_CORES * NUM_SUBCORES
PER_W = B // NW
ROWS_PER_W = 2 * PER_W
CHUNKS = ROWS_PER_W // LANES


def _sc_gather_body(users_hbm, items_hbm, table_hbm, u_out, it_out,
                    idx_v, rows_v, sem):
    wid = lax.axis_index("s") * NUM_CORES + lax.axis_index("c")
    base = wid * PER_W
    pltpu.sync_copy(users_hbm.at[pl.ds(base, PER_W)], idx_v.at[pl.ds(0, PER_W)])
    pltpu.sync_copy(items_hbm.at[pl.ds(base, PER_W)],
                    idx_v.at[pl.ds(PER_W, PER_W)])
    lane = lax.iota(jnp.int32, LANES)

    def chunk_body(c, _):
        vec = idx_v[pl.ds(c * LANES, LANES)]
        for j in range(LANES):
            r = jnp.sum(jnp.where(lane == j, vec, 0))
            pltpu.async_copy(table_hbm.at[pl.ds(r, 1)],
                             rows_v.at[pl.ds(c * LANES + j, 1)], sem)
        return 0

    lax.fori_loop(0, CHUNKS, chunk_body, 0)
    pltpu.make_async_copy(table_hbm.at[pl.ds(0, ROWS_PER_W)], rows_v, sem).wait()
    pltpu.sync_copy(rows_v.at[pl.ds(0, PER_W)], u_out.at[pl.ds(base, PER_W)])
    pltpu.sync_copy(rows_v.at[pl.ds(PER_W, PER_W)],
                    it_out.at[pl.ds(base, PER_W)])


def _make_sc_gather():
    return functools.partial(
        pl.kernel,
        mesh=plsc.VectorSubcoreMesh(core_axis_name="c", subcore_axis_name="s"),
        out_type=(
            jax.ShapeDtypeStruct((B, EMBED_DIM), jnp.float32),
            jax.ShapeDtypeStruct((B, EMBED_DIM), jnp.float32),
        ),
        scratch_types=[
            pltpu.VMEM((ROWS_PER_W,), jnp.int32),
            pltpu.VMEM((ROWS_PER_W, EMBED_DIM), jnp.float32),
            pltpu.SemaphoreType.DMA,
        ],
        compiler_params=pltpu.CompilerParams(needs_layout_passes=False),
    )(_sc_gather_body)


def _mlp_body(u_ref, i_ref, w1_ref, b1_ref, w2_ref, b2_ref,
              w3_ref, b3_ref, w4_ref, out_ref):
    u = u_ref[...].astype(jnp.float32)
    it = i_ref[...].astype(jnp.float32)
    x = (u @ w1_ref[:EMBED_DIM, :] + it @ w1_ref[EMBED_DIM:, :]
         + b1_ref[...].reshape(1, -1))
    x = jnp.maximum(x, 0.0)
    x = jnp.maximum(x @ w2_ref[...] + b2_ref[...].reshape(1, -1), 0.0)
    x = jnp.maximum(x @ w3_ref[...] + b3_ref[...].reshape(1, -1), 0.0)
    out_ref[...] = x @ w4_ref[...]


TCOL = 16384
TGRID = -(-N_USERS // TCOL)


def _transpose_body(tt_ref, out_ref):
    a = tt_ref[...]
    row = lax.broadcasted_iota(jnp.int32, (EMBED_DIM, EMBED_DIM), 0)
    col = lax.broadcasted_iota(jnp.int32, (EMBED_DIM, EMBED_DIM), 1)
    eye = (row == col).astype(jnp.float32)
    out_ref[...] = lax.dot_general(a, eye, (((0,), (0,)), ((), ())),
                                   preferred_element_type=jnp.float32)


def _relayout_table(table_user):
    tt = table_user.T
    return pl.pallas_call(
        _transpose_body,
        grid=(TGRID,),
        in_specs=[pl.BlockSpec((EMBED_DIM, TCOL), lambda i: (0, i))],
        out_specs=pl.BlockSpec((TCOL, EMBED_DIM), lambda i: (i, 0)),
        out_shape=jax.ShapeDtypeStruct((N_USERS, EMBED_DIM), jnp.float32),
        compiler_params=pltpu.CompilerParams(fuse_transposed_lhs_in_matmul=True),
    )(tt)


def kernel(users, items, table_user, W1, b1, W2, b2, W3, b3, W4):
    tbl = _relayout_table(table_user)
    u, it = _make_sc_gather()(users.astype(jnp.int32), items.astype(jnp.int32),
                              tbl)
    score = pl.pallas_call(
        _mlp_body,
        out_shape=jax.ShapeDtypeStruct((B, 1), jnp.float32),
    )(u, it, W1, b1, W2, b2, W3, b3, W4)
    return score


# speedup vs baseline: 2.7618x; 1.0046x over previous
import functools

import jax
import jax.numpy as jnp
from jax import lax
from jax.experimental import pallas as pl
from jax.experimental.pallas import tpu as pltpu
from jax.experimental.pallas import tpu_sc as plsc

N_USERS = 100000
EMBED_DIM = 64
B = 4096

NUM_CORES = 2
NUM_SUBCORES = 16
LANES = 16
NW = NUM_CORES * NUM_SUBCORES
PER_W = B // NW
ROWS_PER_W = 2 * PER_W
CHUNKS = ROWS_PER_W // LANES


def _sc_gather_body(users_hbm, items_hbm, table_hbm, u_out, it_out,
                    idx_v, rows_v, sem):
    wid = lax.axis_index("s") * NUM_CORES + lax.axis_index("c")
    base = wid * PER_W
    pltpu.sync_copy(users_hbm.at[pl.ds(base, PER_W)], idx_v.at[pl.ds(0, PER_W)])
    pltpu.sync_copy(items_hbm.at[pl.ds(base, PER_W)],
                    idx_v.at[pl.ds(PER_W, PER_W)])
    lane = lax.iota(jnp.int32, LANES)

    def chunk_body(c, _):
        vec = idx_v[pl.ds(c * LANES, LANES)]
        for j in range(LANES):
            r = jnp.sum(jnp.where(lane == j, vec, 0))
            pltpu.async_copy(table_hbm.at[pl.ds(r, 1)],
                             rows_v.at[pl.ds(c * LANES + j, 1)], sem)
        return 0

    lax.fori_loop(0, CHUNKS, chunk_body, 0)
    pltpu.make_async_copy(table_hbm.at[pl.ds(0, ROWS_PER_W)], rows_v, sem).wait()
    pltpu.sync_copy(rows_v.at[pl.ds(0, PER_W)], u_out.at[pl.ds(base, PER_W)])
    pltpu.sync_copy(rows_v.at[pl.ds(PER_W, PER_W)],
                    it_out.at[pl.ds(base, PER_W)])


def _make_sc_gather():
    return functools.partial(
        pl.kernel,
        mesh=plsc.VectorSubcoreMesh(core_axis_name="c", subcore_axis_name="s"),
        out_type=(
            jax.ShapeDtypeStruct((B, EMBED_DIM), jnp.float32),
            jax.ShapeDtypeStruct((B, EMBED_DIM), jnp.float32),
        ),
        scratch_types=[
            pltpu.VMEM((ROWS_PER_W,), jnp.int32),
            pltpu.VMEM((ROWS_PER_W, EMBED_DIM), jnp.float32),
            pltpu.SemaphoreType.DMA,
        ],
        compiler_params=pltpu.CompilerParams(needs_layout_passes=False),
    )(_sc_gather_body)


def _mlp_body(u_ref, i_ref, w1_ref, b1_ref, w2_ref, b2_ref,
              w3_ref, b3_ref, w4_ref, out_ref):
    u = u_ref[...].astype(jnp.float32)
    it = i_ref[...].astype(jnp.float32)
    x = (u @ w1_ref[:EMBED_DIM, :] + it @ w1_ref[EMBED_DIM:, :]
         + b1_ref[...].reshape(1, -1))
    x = jnp.maximum(x, 0.0)
    x = jnp.maximum(x @ w2_ref[...] + b2_ref[...].reshape(1, -1), 0.0)
    x = jnp.maximum(x @ w3_ref[...] + b3_ref[...].reshape(1, -1), 0.0)
    out_ref[...] = x @ w4_ref[...]


TCOL = 25088
TGRID = -(-N_USERS // TCOL)


def _transpose_body(tt_ref, out_ref):
    a = tt_ref[...]
    row = lax.broadcasted_iota(jnp.int32, (EMBED_DIM, EMBED_DIM), 0)
    col = lax.broadcasted_iota(jnp.int32, (EMBED_DIM, EMBED_DIM), 1)
    eye = (row == col).astype(jnp.float32)
    out_ref[...] = lax.dot_general(a, eye, (((0,), (0,)), ((), ())),
                                   preferred_element_type=jnp.float32)


def _relayout_table(table_user):
    tt = table_user.T
    return pl.pallas_call(
        _transpose_body,
        grid=(TGRID,),
        in_specs=[pl.BlockSpec((EMBED_DIM, TCOL), lambda i: (0, i))],
        out_specs=pl.BlockSpec((TCOL, EMBED_DIM), lambda i: (i, 0)),
        out_shape=jax.ShapeDtypeStruct((N_USERS, EMBED_DIM), jnp.float32),
        compiler_params=pltpu.CompilerParams(fuse_transposed_lhs_in_matmul=True),
    )(tt)


def kernel(users, items, table_user, W1, b1, W2, b2, W3, b3, W4):
    tbl = _relayout_table(table_user)
    u, it = _make_sc_gather()(users.astype(jnp.int32), items.astype(jnp.int32),
                              tbl)
    score = pl.pallas_call(
        _mlp_body,
        out_shape=jax.ShapeDtypeStruct((B, 1), jnp.float32),
    )(u, it, W1, b1, W2, b2, W3, b3, W4)
    return score
